# Initial kernel scaffold; baseline (speedup 1.0000x reference)
import jax
import jax.numpy as jnp
from jax.experimental import pallas as pl


def kernel(lin_w_question, lin_b_question, bn_gamma_question, bn_beta_question, lin_w_answer, lin_b_answer, bn_gamma_answer, bn_beta_answer, lin_w_concept, lin_b_concept, bn_gamma_concept, bn_beta_concept, c0_k_w_question, c0_k_b_question, c0_q_w_question, c0_q_b_question, c0_v_w_question, c0_v_b_question, c0_alin_w_question, c0_alin_b_question, c0_skip_question, c0_k_w_answer, c0_k_b_answer, c0_q_w_answer, c0_q_b_answer, c0_v_w_answer, c0_v_b_answer, c0_alin_w_answer, c0_alin_b_answer, c0_skip_answer, c0_k_w_concept, c0_k_b_concept, c0_q_w_concept, c0_q_b_concept, c0_v_w_concept, c0_v_b_concept, c0_alin_w_concept, c0_alin_b_concept, c0_skip_concept, c0_arel_question_has_answer, c0_mrel_question_has_answer, c0_prel_question_has_answer, c0_arel_answer_rev_has_question, c0_mrel_answer_rev_has_question, c0_prel_answer_rev_has_question, c0_arel_question_mentions_concept, c0_mrel_question_mentions_concept, c0_prel_question_mentions_concept, c0_arel_concept_rev_mentions_question, c0_mrel_concept_rev_mentions_question, c0_prel_concept_rev_mentions_question, c1_k_w_question, c1_k_b_question, c1_q_w_question, c1_q_b_question, c1_v_w_question, c1_v_b_question, c1_alin_w_question, c1_alin_b_question, c1_skip_question, c1_k_w_answer, c1_k_b_answer, c1_q_w_answer, c1_q_b_answer, c1_v_w_answer, c1_v_b_answer, c1_alin_w_answer, c1_alin_b_answer, c1_skip_answer, c1_k_w_concept, c1_k_b_concept, c1_q_w_concept, c1_q_b_concept, c1_v_w_concept, c1_v_b_concept, c1_alin_w_concept, c1_alin_b_concept, c1_skip_concept, c1_arel_question_has_answer, c1_mrel_question_has_answer, c1_prel_question_has_answer, c1_arel_answer_rev_has_question, c1_mrel_answer_rev_has_question, c1_prel_answer_rev_has_question, c1_arel_question_mentions_concept, c1_mrel_question_mentions_concept, c1_prel_question_mentions_concept, c1_arel_concept_rev_mentions_question, c1_mrel_concept_rev_mentions_question, c1_prel_concept_rev_mentions_question, x_question, x_answer, x_concept, edge_question_has_answer, edge_answer_rev_has_question, edge_question_mentions_concept, edge_concept_rev_mentions_question):
    raise NotImplementedError("write your pallas kernel here")


# speedup vs baseline: 1.5226x; 1.5226x over previous
import functools
import math

import jax
import jax.numpy as jnp
from jax.experimental import pallas as pl
from jax.experimental.pallas import tpu as pltpu

_BF16 = jnp.bfloat16
_SQRT2 = math.sqrt(2.0)

_CH = 512
_H = 8
_HD = 64
_NQ, _NA, _NC = 512, 1024, 768
_NTOT = _NQ + _NA + _NC
_ROWS = {"question": (0, 512), "answer": (512, 1536), "concept": (1536, 2304)}
_NEG = -1e30


def _erf(x):
    a1, a2, a3, a4, a5 = 0.254829592, -0.284496736, 1.421413741, -1.453152027, 1.061405429
    p = 0.3275911
    sgn = jnp.where(x >= 0.0, 1.0, -1.0)
    ax = jnp.abs(x)
    t = 1.0 / (1.0 + p * ax)
    poly = ((((a5 * t + a4) * t + a3) * t + a2) * t + a1) * t
    return sgn * (1.0 - poly * jnp.exp(-ax * ax))


def _gelu_exact(x):
    return 0.5 * x * (1.0 + _erf(x / _SQRT2))


def _dot(a, b):
    return jnp.dot(a, b, preferred_element_type=jnp.float32)


def _dot_nt(a, b):
    return jax.lax.dot_general(a, b, (((1,), (1,)), ((), ())),
                               preferred_element_type=jnp.float32)


def _attend(hb_in, dst, srcs, qw, qb, alin_w, alin_b, alpha, lc, kc, vc, write):
    d0, d1 = _ROWS[dst]
    hd = hb_in[d0:d1]
    q = (_dot(hd, qw[...]) + qb[...]).astype(_BF16)
    off = 0
    for (s0, s1), kw, kb, vw, vb in srcs:
        hs = hb_in[s0:s1]
        ns = s1 - s0
        kc[off:off + ns] = (_dot(hs, kw[...]) + kb[...]).astype(_BF16)
        vc[off:off + ns] = (_dot(hs, vw[...]) + vb[...]).astype(_BF16)
        off += ns
    spans = []
    o = 0
    for (s0, s1), _, _, _, _ in srcs:
        spans.append((o, o + (s1 - s0)))
        o += s1 - s0
    lcv = lc[...]
    outs = []
    for h in range(_H):
        sl = slice(h * _HD, (h + 1) * _HD)
        t = _dot_nt(q[:, sl], kc[:, sl]) + lcv
        parts = []
        for o0, o1 in spans:
            te = t[:, o0:o1]
            rm = jnp.max(te, axis=-1, keepdims=True)
            ok = rm > -1e29
            w = jnp.exp(te - rm)
            denom = jnp.sum(w, axis=-1, keepdims=True)
            inv = jnp.where(ok, 1.0 / denom, 0.0)
            parts.append(w * inv)
        wn = parts[0] if len(parts) == 1 else jnp.concatenate(parts, axis=-1)
        outs.append(_dot(wn.astype(_BF16), vc[:off, sl]))
    att = jnp.concatenate(outs, axis=-1)
    g = _gelu_exact(att).astype(_BF16)
    y = _dot(g, alin_w[...]) + alin_b[...]
    a = alpha[...]
    write(a * y + (1.0 - a) * hd.astype(jnp.float32))


def _layer(hb_in, lw, lcs, writers):
    specs = [
        ("question", 1792,
         [(_ROWS["answer"],) + lw["k_rev_has"] + lw["v_rev_has"],
          (_ROWS["concept"],) + lw["k_rev_mentions"] + lw["v_rev_mentions"]]),
        ("answer", 512, [(_ROWS["question"],) + lw["k_has"] + lw["v_has"]]),
        ("concept", 512, [(_ROWS["question"],) + lw["k_mentions"] + lw["v_mentions"]]),
    ]
    for dst, ns_tot, srcs in specs:
        fn = functools.partial(
            _attend, hb_in, dst,
            [(rng, kw, kb, vw, vb) for rng, kw, kb, vw, vb in srcs],
            lw["q_" + dst][0], lw["q_" + dst][1],
            lw["alin_" + dst][0], lw["alin_" + dst][1],
            lw["alpha_" + dst], lcs[dst])
        pl.run_scoped(functools.partial(lambda f, w, kc, vc: f(kc, vc, w),
                                        fn, writers[dst]),
                      pltpu.VMEM((ns_tot, _CH), _BF16),
                      pltpu.VMEM((ns_tot, _CH), _BF16))


def _body(nin, *args):
    refs = list(args[:nin])
    out_q, out_a, out_c = args[nin:nin + 3]
    hb0, hb1 = args[nin + 3:]
    it = iter(refs)

    def nxt():
        return next(it)

    xs = {"question": nxt(), "answer": nxt(), "concept": nxt()}
    lin = {t: (nxt(), nxt(), nxt(), nxt())
           for t in ("question", "answer", "concept")}

    layers = []
    for _ in range(2):
        lw = {}
        for t in ("question", "answer", "concept"):
            lw["q_" + t] = (nxt(), nxt())
        for et in ("has", "rev_has", "mentions", "rev_mentions"):
            lw["k_" + et] = (nxt(), nxt())
            lw["v_" + et] = (nxt(), nxt())
        for t in ("question", "answer", "concept"):
            lw["alin_" + t] = (nxt(), nxt())
        for t in ("question", "answer", "concept"):
            lw["alpha_" + t] = nxt()
        layers.append(lw)

    lcs = {"question": nxt(), "answer": nxt(), "concept": nxt()}

    for t in ("question", "answer", "concept"):
        r0, r1 = _ROWS[t]
        w, b, gamma, beta = lin[t]
        y = _dot(xs[t][...], w[...]) + b[...]
        y = jnp.maximum(y, 0.0)
        n = r1 - r0
        mean = jnp.sum(y, axis=0, keepdims=True) * (1.0 / n)
        yc = y - mean
        var = jnp.sum(yc * yc, axis=0, keepdims=True) * (1.0 / n)
        y = yc * jax.lax.rsqrt(var + 1e-5) * gamma[...] + beta[...]
        hb0[r0:r1] = y.astype(_BF16)

    def w0(dst):
        def wr(v):
            r0, r1 = _ROWS[dst]
            hb1[r0:r1] = v.astype(_BF16)
        return wr

    _layer(hb0, layers[0], lcs,
           {d: w0(d) for d in ("question", "answer", "concept")})

    outs = {"question": out_q, "answer": out_a, "concept": out_c}

    def w1(dst):
        def wr(v):
            outs[dst][...] = v
        return wr

    _layer(hb1, layers[1], lcs,
           {d: w1(d) for d in ("question", "answer", "concept")})


def _fold(w, b, rel):
    wf = jnp.einsum("chd,hde->che", w.reshape(_CH, _H, _HD), rel,
                    preferred_element_type=jnp.float32).reshape(_CH, _CH)
    bf = jnp.einsum("hd,hde->he", b.reshape(_H, _HD), rel,
                    preferred_element_type=jnp.float32).reshape(1, _CH)
    return wf.astype(_BF16), bf


def _logcnt(shape, scatters):
    cnt = jnp.zeros(shape, jnp.float32)
    for ei, col_off in scatters:
        cnt = cnt.at[ei[1], ei[0] + col_off].add(1.0)
    return jnp.where(cnt > 0.0, jnp.log(cnt), _NEG)


def kernel(lin_w_question, lin_b_question, bn_gamma_question, bn_beta_question, lin_w_answer, lin_b_answer, bn_gamma_answer, bn_beta_answer, lin_w_concept, lin_b_concept, bn_gamma_concept, bn_beta_concept, c0_k_w_question, c0_k_b_question, c0_q_w_question, c0_q_b_question, c0_v_w_question, c0_v_b_question, c0_alin_w_question, c0_alin_b_question, c0_skip_question, c0_k_w_answer, c0_k_b_answer, c0_q_w_answer, c0_q_b_answer, c0_v_w_answer, c0_v_b_answer, c0_alin_w_answer, c0_alin_b_answer, c0_skip_answer, c0_k_w_concept, c0_k_b_concept, c0_q_w_concept, c0_q_b_concept, c0_v_w_concept, c0_v_b_concept, c0_alin_w_concept, c0_alin_b_concept, c0_skip_concept, c0_arel_question_has_answer, c0_mrel_question_has_answer, c0_prel_question_has_answer, c0_arel_answer_rev_has_question, c0_mrel_answer_rev_has_question, c0_prel_answer_rev_has_question, c0_arel_question_mentions_concept, c0_mrel_question_mentions_concept, c0_prel_question_mentions_concept, c0_arel_concept_rev_mentions_question, c0_mrel_concept_rev_mentions_question, c0_prel_concept_rev_mentions_question, c1_k_w_question, c1_k_b_question, c1_q_w_question, c1_q_b_question, c1_v_w_question, c1_v_b_question, c1_alin_w_question, c1_alin_b_question, c1_skip_question, c1_k_w_answer, c1_k_b_answer, c1_q_w_answer, c1_q_b_answer, c1_v_w_answer, c1_v_b_answer, c1_alin_w_answer, c1_alin_b_answer, c1_skip_answer, c1_k_w_concept, c1_k_b_concept, c1_q_w_concept, c1_q_b_concept, c1_v_w_concept, c1_v_b_concept, c1_alin_w_concept, c1_alin_b_concept, c1_skip_concept, c1_arel_question_has_answer, c1_mrel_question_has_answer, c1_prel_question_has_answer, c1_arel_answer_rev_has_question, c1_mrel_answer_rev_has_question, c1_prel_answer_rev_has_question, c1_arel_question_mentions_concept, c1_mrel_question_mentions_concept, c1_prel_question_mentions_concept, c1_arel_concept_rev_mentions_question, c1_mrel_concept_rev_mentions_question, c1_prel_concept_rev_mentions_question, x_question, x_answer, x_concept, edge_question_has_answer, edge_answer_rev_has_question, edge_question_mentions_concept, edge_concept_rev_mentions_question):
    c0 = {
        "k_w": {"question": c0_k_w_question, "answer": c0_k_w_answer, "concept": c0_k_w_concept},
        "k_b": {"question": c0_k_b_question, "answer": c0_k_b_answer, "concept": c0_k_b_concept},
        "q_w": {"question": c0_q_w_question, "answer": c0_q_w_answer, "concept": c0_q_w_concept},
        "q_b": {"question": c0_q_b_question, "answer": c0_q_b_answer, "concept": c0_q_b_concept},
        "v_w": {"question": c0_v_w_question, "answer": c0_v_w_answer, "concept": c0_v_w_concept},
        "v_b": {"question": c0_v_b_question, "answer": c0_v_b_answer, "concept": c0_v_b_concept},
        "alin_w": {"question": c0_alin_w_question, "answer": c0_alin_w_answer, "concept": c0_alin_w_concept},
        "alin_b": {"question": c0_alin_b_question, "answer": c0_alin_b_answer, "concept": c0_alin_b_concept},
        "skip": {"question": c0_skip_question, "answer": c0_skip_answer, "concept": c0_skip_concept},
        "arel": {"has": c0_arel_question_has_answer, "rev_has": c0_arel_answer_rev_has_question,
                 "mentions": c0_arel_question_mentions_concept, "rev_mentions": c0_arel_concept_rev_mentions_question},
        "mrel": {"has": c0_mrel_question_has_answer, "rev_has": c0_mrel_answer_rev_has_question,
                 "mentions": c0_mrel_question_mentions_concept, "rev_mentions": c0_mrel_concept_rev_mentions_question},
        "prel": {"has": c0_prel_question_has_answer, "rev_has": c0_prel_answer_rev_has_question,
                 "mentions": c0_prel_question_mentions_concept, "rev_mentions": c0_prel_concept_rev_mentions_question},
    }
    c1 = {
        "k_w": {"question": c1_k_w_question, "answer": c1_k_w_answer, "concept": c1_k_w_concept},
        "k_b": {"question": c1_k_b_question, "answer": c1_k_b_answer, "concept": c1_k_b_concept},
        "q_w": {"question": c1_q_w_question, "answer": c1_q_w_answer, "concept": c1_q_w_concept},
        "q_b": {"question": c1_q_b_question, "answer": c1_q_b_answer, "concept": c1_q_b_concept},
        "v_w": {"question": c1_v_w_question, "answer": c1_v_w_answer, "concept": c1_v_w_concept},
        "v_b": {"question": c1_v_b_question, "answer": c1_v_b_answer, "concept": c1_v_b_concept},
        "alin_w": {"question": c1_alin_w_question, "answer": c1_alin_w_answer, "concept": c1_alin_w_concept},
        "alin_b": {"question": c1_alin_b_question, "answer": c1_alin_b_answer, "concept": c1_alin_b_concept},
        "skip": {"question": c1_skip_question, "answer": c1_skip_answer, "concept": c1_skip_concept},
        "arel": {"has": c1_arel_question_has_answer, "rev_has": c1_arel_answer_rev_has_question,
                 "mentions": c1_arel_question_mentions_concept, "rev_mentions": c1_arel_concept_rev_mentions_question},
        "mrel": {"has": c1_mrel_question_has_answer, "rev_has": c1_mrel_answer_rev_has_question,
                 "mentions": c1_mrel_question_mentions_concept, "rev_mentions": c1_mrel_concept_rev_mentions_question},
        "prel": {"has": c1_prel_question_has_answer, "rev_has": c1_prel_answer_rev_has_question,
                 "mentions": c1_prel_question_mentions_concept, "rev_mentions": c1_prel_concept_rev_mentions_question},
    }
    et_src = {"has": "question", "rev_has": "answer",
              "mentions": "question", "rev_mentions": "concept"}

    ins = [x_question.astype(_BF16), x_answer.astype(_BF16), x_concept.astype(_BF16)]
    lin_w = {"question": lin_w_question, "answer": lin_w_answer, "concept": lin_w_concept}
    lin_b = {"question": lin_b_question, "answer": lin_b_answer, "concept": lin_b_concept}
    bn_g = {"question": bn_gamma_question, "answer": bn_gamma_answer, "concept": bn_gamma_concept}
    bn_b = {"question": bn_beta_question, "answer": bn_beta_answer, "concept": bn_beta_concept}
    for t in ("question", "answer", "concept"):
        ins += [lin_w[t].astype(_BF16), lin_b[t], bn_g[t], bn_b[t]]

    for cl in (c0, c1):
        for t in ("question", "answer", "concept"):
            ins += [cl["q_w"][t].astype(_BF16), cl["q_b"][t]]
        for et in ("has", "rev_has", "mentions", "rev_mentions"):
            s = et_src[et]
            ka = cl["arel"][et] * (cl["prel"][et] / math.sqrt(_HD))[:, None, None]
            kw, kb = _fold(cl["k_w"][s], cl["k_b"][s], ka)
            vw, vb = _fold(cl["v_w"][s], cl["v_b"][s], cl["mrel"][et])
            ins += [kw, kb, vw, vb]
        for t in ("question", "answer", "concept"):
            ins += [cl["alin_w"][t].astype(_BF16), cl["alin_b"][t]]
        for t in ("question", "answer", "concept"):
            ins.append(jnp.broadcast_to(jax.nn.sigmoid(cl["skip"][t]),
                                        (1, _CH)).astype(jnp.float32))

    ins.append(_logcnt((_NQ, _NA + _NC),
                       [(edge_answer_rev_has_question, 0),
                        (edge_concept_rev_mentions_question, _NA)]))
    ins.append(_logcnt((_NA, _NQ), [(edge_question_has_answer, 0)]))
    ins.append(_logcnt((_NC, _NQ), [(edge_question_mentions_concept, 0)]))

    nin = len(ins)
    out = pl.pallas_call(
        functools.partial(_body, nin),
        out_shape=(jax.ShapeDtypeStruct((_NQ, _CH), jnp.float32),
                   jax.ShapeDtypeStruct((_NA, _CH), jnp.float32),
                   jax.ShapeDtypeStruct((_NC, _CH), jnp.float32)),
        in_specs=[pl.BlockSpec(memory_space=pltpu.MemorySpace.VMEM)] * nin,
        out_specs=(pl.BlockSpec(memory_space=pltpu.MemorySpace.VMEM),) * 3,
        scratch_shapes=[pltpu.VMEM((_NTOT, _CH), _BF16),
                        pltpu.VMEM((_NTOT, _CH), _BF16)],
        compiler_params=pltpu.CompilerParams(
            vmem_limit_bytes=56 * 1024 * 1024),
    )(*ins)
    return {"question": out[0], "answer": out[1], "concept": out[2]}


# speedup vs baseline: 2.8928x; 1.8999x over previous
import functools
import math

import jax
import jax.numpy as jnp
from jax.experimental import pallas as pl
from jax.experimental.pallas import tpu as pltpu

_BF16 = jnp.bfloat16
_SQRT2 = math.sqrt(2.0)

_CH = 512
_H = 8
_HD = 64
_NQ, _NA, _NC = 512, 1024, 768
_NTOT = _NQ + _NA + _NC
_ROWS = {"question": (0, 512), "answer": (512, 1536), "concept": (1536, 2304)}
_NEG = -1e30


def _erf(x):
    a1, a2, a3, a4, a5 = 0.254829592, -0.284496736, 1.421413741, -1.453152027, 1.061405429
    p = 0.3275911
    sgn = jnp.where(x >= 0.0, 1.0, -1.0)
    ax = jnp.abs(x)
    t = 1.0 / (1.0 + p * ax)
    poly = ((((a5 * t + a4) * t + a3) * t + a2) * t + a1) * t
    return sgn * (1.0 - poly * jnp.exp(-ax * ax))


def _gelu_exact(x):
    return 0.5 * x * (1.0 + _erf(x / _SQRT2))


def _dot(a, b):
    return jnp.dot(a, b, preferred_element_type=jnp.float32)


def _dot_nt(a, b):
    return jax.lax.dot_general(a, b, (((1,), (1,)), ((), ())),
                               preferred_element_type=jnp.float32)


def _attend(hb_in, dst, srcs, qw, qb, alin_w, alin_b, alpha, lc, kc, vc, write):
    d0, d1 = _ROWS[dst]
    hd = hb_in[d0:d1]
    q = (_dot(hd, qw[...]) + qb[...]).astype(_BF16)
    off = 0
    for (s0, s1), kw, kb, vw, vb in srcs:
        hs = hb_in[s0:s1]
        ns = s1 - s0
        kc[off:off + ns] = (_dot(hs, kw[...]) + kb[...]).astype(_BF16)
        vc[off:off + ns] = (_dot(hs, vw[...]) + vb[...]).astype(_BF16)
        off += ns
    spans = []
    o = 0
    for (s0, s1), _, _, _, _ in srcs:
        spans.append((o, o + (s1 - s0)))
        o += s1 - s0
    lcv = lc[...]
    outs = []
    for h in range(_H):
        sl = slice(h * _HD, (h + 1) * _HD)
        t = _dot_nt(q[:, sl], kc[:, sl]) + lcv
        parts = []
        for o0, o1 in spans:
            te = t[:, o0:o1]
            rm = jnp.max(te, axis=-1, keepdims=True)
            ok = rm > -1e29
            w = jnp.exp(te - rm)
            denom = jnp.sum(w, axis=-1, keepdims=True)
            inv = jnp.where(ok, 1.0 / denom, 0.0)
            parts.append(w * inv)
        wn = parts[0] if len(parts) == 1 else jnp.concatenate(parts, axis=-1)
        outs.append(_dot(wn.astype(_BF16), vc[:off, sl]))
    att = jnp.concatenate(outs, axis=-1)
    g = _gelu_exact(att).astype(_BF16)
    y = _dot(g, alin_w[...]) + alin_b[...]
    a = alpha[...]
    write(a * y + (1.0 - a) * hd.astype(jnp.float32))


def _layer(hb_in, lw, lcs, writers):
    specs = [
        ("question", 1792,
         [(_ROWS["answer"],) + lw["k_rev_has"] + lw["v_rev_has"],
          (_ROWS["concept"],) + lw["k_rev_mentions"] + lw["v_rev_mentions"]]),
        ("answer", 512, [(_ROWS["question"],) + lw["k_has"] + lw["v_has"]]),
        ("concept", 512, [(_ROWS["question"],) + lw["k_mentions"] + lw["v_mentions"]]),
    ]
    for dst, ns_tot, srcs in specs:
        fn = functools.partial(
            _attend, hb_in, dst,
            [(rng, kw, kb, vw, vb) for rng, kw, kb, vw, vb in srcs],
            lw["q_" + dst][0], lw["q_" + dst][1],
            lw["alin_" + dst][0], lw["alin_" + dst][1],
            lw["alpha_" + dst], lcs[dst])
        pl.run_scoped(functools.partial(lambda f, w, kc, vc: f(kc, vc, w),
                                        fn, writers[dst]),
                      pltpu.VMEM((ns_tot, _CH), _BF16),
                      pltpu.VMEM((ns_tot, _CH), _BF16))


def _build_lc(e_ref, nd, ns, out_ref, col0):
    ne = e_ref.shape[1]

    def f(a_ref, b_ref):
        a_ref[...] = (jax.lax.broadcasted_iota(jnp.int32, (nd, ne), 0)
                      == e_ref[1:2, :]).astype(_BF16)
        b_ref[...] = (jax.lax.broadcasted_iota(jnp.int32, (ns, ne), 0)
                      == e_ref[0:1, :]).astype(_BF16)
        cnt = _dot_nt(a_ref[...], b_ref[...])
        out_ref[:, col0:col0 + ns] = jnp.where(cnt > 0.0, jnp.log(cnt), _NEG)

    pl.run_scoped(f, pltpu.VMEM((nd, ne), _BF16), pltpu.VMEM((ns, ne), _BF16))


def _body(nin, *args):
    refs = list(args[:nin])
    out_q, out_a, out_c = args[nin:nin + 3]
    hb0, hb1, lc_q, lc_a, lc_c = args[nin + 3:]
    it = iter(refs)

    def nxt():
        return next(it)

    xs = {"question": nxt(), "answer": nxt(), "concept": nxt()}
    lin = {t: (nxt(), nxt(), nxt(), nxt())
           for t in ("question", "answer", "concept")}

    layers = []
    for _ in range(2):
        lw = {}
        for t in ("question", "answer", "concept"):
            lw["q_" + t] = (nxt(), nxt())
        for et in ("has", "rev_has", "mentions", "rev_mentions"):
            lw["k_" + et] = (nxt(), nxt())
            lw["v_" + et] = (nxt(), nxt())
        for t in ("question", "answer", "concept"):
            lw["alin_" + t] = (nxt(), nxt())
        for t in ("question", "answer", "concept"):
            lw["alpha_" + t] = nxt()
        layers.append(lw)

    e_has, e_rev_has, e_mentions, e_rev_mentions = nxt(), nxt(), nxt(), nxt()
    _build_lc(e_rev_has, _NQ, _NA, lc_q, 0)
    _build_lc(e_rev_mentions, _NQ, _NC, lc_q, _NA)
    _build_lc(e_has, _NA, _NQ, lc_a, 0)
    _build_lc(e_mentions, _NC, _NQ, lc_c, 0)
    lcs = {"question": lc_q, "answer": lc_a, "concept": lc_c}

    for t in ("question", "answer", "concept"):
        r0, r1 = _ROWS[t]
        w, b, gamma, beta = lin[t]
        y = _dot(xs[t][...], w[...]) + b[...]
        y = jnp.maximum(y, 0.0)
        n = r1 - r0
        mean = jnp.sum(y, axis=0, keepdims=True) * (1.0 / n)
        yc = y - mean
        var = jnp.sum(yc * yc, axis=0, keepdims=True) * (1.0 / n)
        y = yc * jax.lax.rsqrt(var + 1e-5) * gamma[...] + beta[...]
        hb0[r0:r1] = y.astype(_BF16)

    def w0(dst):
        def wr(v):
            r0, r1 = _ROWS[dst]
            hb1[r0:r1] = v.astype(_BF16)
        return wr

    _layer(hb0, layers[0], lcs,
           {d: w0(d) for d in ("question", "answer", "concept")})

    outs = {"question": out_q, "answer": out_a, "concept": out_c}

    def w1(dst):
        def wr(v):
            outs[dst][...] = v
        return wr

    _layer(hb1, layers[1], lcs,
           {d: w1(d) for d in ("question", "answer", "concept")})


def _fold(w, b, rel):
    wf = jnp.einsum("chd,hde->che", w.reshape(_CH, _H, _HD), rel,
                    preferred_element_type=jnp.float32).reshape(_CH, _CH)
    bf = jnp.einsum("hd,hde->he", b.reshape(_H, _HD), rel,
                    preferred_element_type=jnp.float32).reshape(1, _CH)
    return wf.astype(_BF16), bf


def kernel(lin_w_question, lin_b_question, bn_gamma_question, bn_beta_question, lin_w_answer, lin_b_answer, bn_gamma_answer, bn_beta_answer, lin_w_concept, lin_b_concept, bn_gamma_concept, bn_beta_concept, c0_k_w_question, c0_k_b_question, c0_q_w_question, c0_q_b_question, c0_v_w_question, c0_v_b_question, c0_alin_w_question, c0_alin_b_question, c0_skip_question, c0_k_w_answer, c0_k_b_answer, c0_q_w_answer, c0_q_b_answer, c0_v_w_answer, c0_v_b_answer, c0_alin_w_answer, c0_alin_b_answer, c0_skip_answer, c0_k_w_concept, c0_k_b_concept, c0_q_w_concept, c0_q_b_concept, c0_v_w_concept, c0_v_b_concept, c0_alin_w_concept, c0_alin_b_concept, c0_skip_concept, c0_arel_question_has_answer, c0_mrel_question_has_answer, c0_prel_question_has_answer, c0_arel_answer_rev_has_question, c0_mrel_answer_rev_has_question, c0_prel_answer_rev_has_question, c0_arel_question_mentions_concept, c0_mrel_question_mentions_concept, c0_prel_question_mentions_concept, c0_arel_concept_rev_mentions_question, c0_mrel_concept_rev_mentions_question, c0_prel_concept_rev_mentions_question, c1_k_w_question, c1_k_b_question, c1_q_w_question, c1_q_b_question, c1_v_w_question, c1_v_b_question, c1_alin_w_question, c1_alin_b_question, c1_skip_question, c1_k_w_answer, c1_k_b_answer, c1_q_w_answer, c1_q_b_answer, c1_v_w_answer, c1_v_b_answer, c1_alin_w_answer, c1_alin_b_answer, c1_skip_answer, c1_k_w_concept, c1_k_b_concept, c1_q_w_concept, c1_q_b_concept, c1_v_w_concept, c1_v_b_concept, c1_alin_w_concept, c1_alin_b_concept, c1_skip_concept, c1_arel_question_has_answer, c1_mrel_question_has_answer, c1_prel_question_has_answer, c1_arel_answer_rev_has_question, c1_mrel_answer_rev_has_question, c1_prel_answer_rev_has_question, c1_arel_question_mentions_concept, c1_mrel_question_mentions_concept, c1_prel_question_mentions_concept, c1_arel_concept_rev_mentions_question, c1_mrel_concept_rev_mentions_question, c1_prel_concept_rev_mentions_question, x_question, x_answer, x_concept, edge_question_has_answer, edge_answer_rev_has_question, edge_question_mentions_concept, edge_concept_rev_mentions_question):
    c0 = {
        "k_w": {"question": c0_k_w_question, "answer": c0_k_w_answer, "concept": c0_k_w_concept},
        "k_b": {"question": c0_k_b_question, "answer": c0_k_b_answer, "concept": c0_k_b_concept},
        "q_w": {"question": c0_q_w_question, "answer": c0_q_w_answer, "concept": c0_q_w_concept},
        "q_b": {"question": c0_q_b_question, "answer": c0_q_b_answer, "concept": c0_q_b_concept},
        "v_w": {"question": c0_v_w_question, "answer": c0_v_w_answer, "concept": c0_v_w_concept},
        "v_b": {"question": c0_v_b_question, "answer": c0_v_b_answer, "concept": c0_v_b_concept},
        "alin_w": {"question": c0_alin_w_question, "answer": c0_alin_w_answer, "concept": c0_alin_w_concept},
        "alin_b": {"question": c0_alin_b_question, "answer": c0_alin_b_answer, "concept": c0_alin_b_concept},
        "skip": {"question": c0_skip_question, "answer": c0_skip_answer, "concept": c0_skip_concept},
        "arel": {"has": c0_arel_question_has_answer, "rev_has": c0_arel_answer_rev_has_question,
                 "mentions": c0_arel_question_mentions_concept, "rev_mentions": c0_arel_concept_rev_mentions_question},
        "mrel": {"has": c0_mrel_question_has_answer, "rev_has": c0_mrel_answer_rev_has_question,
                 "mentions": c0_mrel_question_mentions_concept, "rev_mentions": c0_mrel_concept_rev_mentions_question},
        "prel": {"has": c0_prel_question_has_answer, "rev_has": c0_prel_answer_rev_has_question,
                 "mentions": c0_prel_question_mentions_concept, "rev_mentions": c0_prel_concept_rev_mentions_question},
    }
    c1 = {
        "k_w": {"question": c1_k_w_question, "answer": c1_k_w_answer, "concept": c1_k_w_concept},
        "k_b": {"question": c1_k_b_question, "answer": c1_k_b_answer, "concept": c1_k_b_concept},
        "q_w": {"question": c1_q_w_question, "answer": c1_q_w_answer, "concept": c1_q_w_concept},
        "q_b": {"question": c1_q_b_question, "answer": c1_q_b_answer, "concept": c1_q_b_concept},
        "v_w": {"question": c1_v_w_question, "answer": c1_v_w_answer, "concept": c1_v_w_concept},
        "v_b": {"question": c1_v_b_question, "answer": c1_v_b_answer, "concept": c1_v_b_concept},
        "alin_w": {"question": c1_alin_w_question, "answer": c1_alin_w_answer, "concept": c1_alin_w_concept},
        "alin_b": {"question": c1_alin_b_question, "answer": c1_alin_b_answer, "concept": c1_alin_b_concept},
        "skip": {"question": c1_skip_question, "answer": c1_skip_answer, "concept": c1_skip_concept},
        "arel": {"has": c1_arel_question_has_answer, "rev_has": c1_arel_answer_rev_has_question,
                 "mentions": c1_arel_question_mentions_concept, "rev_mentions": c1_arel_concept_rev_mentions_question},
        "mrel": {"has": c1_mrel_question_has_answer, "rev_has": c1_mrel_answer_rev_has_question,
                 "mentions": c1_mrel_question_mentions_concept, "rev_mentions": c1_mrel_concept_rev_mentions_question},
        "prel": {"has": c1_prel_question_has_answer, "rev_has": c1_prel_answer_rev_has_question,
                 "mentions": c1_prel_question_mentions_concept, "rev_mentions": c1_prel_concept_rev_mentions_question},
    }
    et_src = {"has": "question", "rev_has": "answer",
              "mentions": "question", "rev_mentions": "concept"}

    ins = [x_question.astype(_BF16), x_answer.astype(_BF16), x_concept.astype(_BF16)]
    lin_w = {"question": lin_w_question, "answer": lin_w_answer, "concept": lin_w_concept}
    lin_b = {"question": lin_b_question, "answer": lin_b_answer, "concept": lin_b_concept}
    bn_g = {"question": bn_gamma_question, "answer": bn_gamma_answer, "concept": bn_gamma_concept}
    bn_b = {"question": bn_beta_question, "answer": bn_beta_answer, "concept": bn_beta_concept}
    for t in ("question", "answer", "concept"):
        ins += [lin_w[t].astype(_BF16), lin_b[t], bn_g[t], bn_b[t]]

    for cl in (c0, c1):
        for t in ("question", "answer", "concept"):
            ins += [cl["q_w"][t].astype(_BF16), cl["q_b"][t]]
        for et in ("has", "rev_has", "mentions", "rev_mentions"):
            s = et_src[et]
            ka = cl["arel"][et] * (cl["prel"][et] / math.sqrt(_HD))[:, None, None]
            kw, kb = _fold(cl["k_w"][s], cl["k_b"][s], ka)
            vw, vb = _fold(cl["v_w"][s], cl["v_b"][s], cl["mrel"][et])
            ins += [kw, kb, vw, vb]
        for t in ("question", "answer", "concept"):
            ins += [cl["alin_w"][t].astype(_BF16), cl["alin_b"][t]]
        for t in ("question", "answer", "concept"):
            ins.append(jnp.broadcast_to(jax.nn.sigmoid(cl["skip"][t]),
                                        (1, _CH)).astype(jnp.float32))

    ins += [edge_question_has_answer, edge_answer_rev_has_question,
            edge_question_mentions_concept, edge_concept_rev_mentions_question]

    nin = len(ins)
    out = pl.pallas_call(
        functools.partial(_body, nin),
        out_shape=(jax.ShapeDtypeStruct((_NQ, _CH), jnp.float32),
                   jax.ShapeDtypeStruct((_NA, _CH), jnp.float32),
                   jax.ShapeDtypeStruct((_NC, _CH), jnp.float32)),
        in_specs=[pl.BlockSpec(memory_space=pltpu.MemorySpace.VMEM)] * nin,
        out_specs=(pl.BlockSpec(memory_space=pltpu.MemorySpace.VMEM),) * 3,
        scratch_shapes=[pltpu.VMEM((_NTOT, _CH), _BF16),
                        pltpu.VMEM((_NTOT, _CH), _BF16),
                        pltpu.VMEM((_NQ, _NA + _NC), jnp.float32),
                        pltpu.VMEM((_NA, _NQ), jnp.float32),
                        pltpu.VMEM((_NC, _NQ), jnp.float32)],
        compiler_params=pltpu.CompilerParams(
            vmem_limit_bytes=56 * 1024 * 1024),
    )(*ins)
    return {"question": out[0], "answer": out[1], "concept": out[2]}


# speedup vs baseline: 3.0003x; 1.0372x over previous
import functools
import math

import jax
import jax.numpy as jnp
from jax.experimental import pallas as pl
from jax.experimental.pallas import tpu as pltpu

_BF16 = jnp.bfloat16
_SQRT2 = math.sqrt(2.0)

_CH = 512
_H = 8
_HD = 64
_NQ, _NA, _NC = 512, 1024, 768
_NTOT = _NQ + _NA + _NC
_ROWS = {"question": (0, 512), "answer": (512, 1536), "concept": (1536, 2304)}
_NEG = -1e30


def _erf(x):
    a1, a2, a3, a4, a5 = 0.254829592, -0.284496736, 1.421413741, -1.453152027, 1.061405429
    p = 0.3275911
    sgn = jnp.where(x >= 0.0, 1.0, -1.0)
    ax = jnp.abs(x)
    t = 1.0 / (1.0 + p * ax)
    poly = ((((a5 * t + a4) * t + a3) * t + a2) * t + a1) * t
    return sgn * (1.0 - poly * jnp.exp(-ax * ax))


def _gelu_exact(x):
    return 0.5 * x * (1.0 + _erf(x / _SQRT2))


def _dot(a, b):
    return jnp.dot(a, b, preferred_element_type=jnp.float32)


def _dot_nt(a, b):
    return jax.lax.dot_general(a, b, (((1,), (1,)), ((), ())),
                               preferred_element_type=jnp.float32)


def _attend(hb_in, dst, srcs, qw, qb, alin_w, alin_b, alpha, lc, kc, vc, write):
    d0, d1 = _ROWS[dst]
    hd = hb_in[d0:d1]
    q = (_dot(hd, qw[...]) + qb[...]).astype(_BF16)
    off = 0
    for (s0, s1), kw, kb, vw, vb in srcs:
        hs = hb_in[s0:s1]
        ns = s1 - s0
        kc[off:off + ns] = (_dot(hs, kw[...]) + kb[...]).astype(_BF16)
        vc[off:off + ns] = (_dot(hs, vw[...]) + vb[...]).astype(_BF16)
        off += ns
    spans = []
    o = 0
    for (s0, s1), _, _, _, _ in srcs:
        spans.append((o, o + (s1 - s0)))
        o += s1 - s0
    lcv = lc[...]
    outs = []
    for h in range(_H):
        sl = slice(h * _HD, (h + 1) * _HD)
        t = _dot_nt(q[:, sl], kc[:, sl]) + lcv
        parts = []
        for o0, o1 in spans:
            te = t[:, o0:o1]
            rm = jnp.max(te, axis=-1, keepdims=True)
            ok = rm > -1e29
            w = jnp.exp(te - rm)
            denom = jnp.sum(w, axis=-1, keepdims=True)
            inv = jnp.where(ok, 1.0 / denom, 0.0)
            parts.append(w * inv)
        wn = parts[0] if len(parts) == 1 else jnp.concatenate(parts, axis=-1)
        outs.append(_dot(wn.astype(_BF16), vc[:off, sl]))
    att = jnp.concatenate(outs, axis=-1)
    g = _gelu_exact(att).astype(_BF16)
    y = _dot(g, alin_w[...]) + alin_b[...]
    a = alpha[...]
    write(a * y + (1.0 - a) * hd.astype(jnp.float32))


def _layer(hb_in, lw, lcs, writers):
    specs = [
        ("question", 1792,
         [(_ROWS["answer"],) + lw["k_rev_has"] + lw["v_rev_has"],
          (_ROWS["concept"],) + lw["k_rev_mentions"] + lw["v_rev_mentions"]]),
        ("answer", 512, [(_ROWS["question"],) + lw["k_has"] + lw["v_has"]]),
        ("concept", 512, [(_ROWS["question"],) + lw["k_mentions"] + lw["v_mentions"]]),
    ]
    for dst, ns_tot, srcs in specs:
        fn = functools.partial(
            _attend, hb_in, dst,
            [(rng, kw, kb, vw, vb) for rng, kw, kb, vw, vb in srcs],
            lw["q_" + dst][0], lw["q_" + dst][1],
            lw["alin_" + dst][0], lw["alin_" + dst][1],
            lw["alpha_" + dst], lcs[dst])
        pl.run_scoped(functools.partial(lambda f, w, kc, vc: f(kc, vc, w),
                                        fn, writers[dst]),
                      pltpu.VMEM((ns_tot, _CH), _BF16),
                      pltpu.VMEM((ns_tot, _CH), _BF16))


def _build_lc(e_ref, nd, ns, out_ref, col0):
    ne = e_ref.shape[1]
    dt = jnp.float8_e4m3fn

    def f(a_ref, b_ref):
        a_ref[...] = (jax.lax.broadcasted_iota(jnp.int32, (nd, ne), 0)
                      == e_ref[1:2, :]).astype(dt)
        b_ref[...] = (jax.lax.broadcasted_iota(jnp.int32, (ns, ne), 0)
                      == e_ref[0:1, :]).astype(dt)
        cnt = _dot_nt(a_ref[...], b_ref[...])
        out_ref[:, col0:col0 + ns] = jnp.where(cnt > 0.0, jnp.log(cnt), _NEG)

    pl.run_scoped(f, pltpu.VMEM((nd, ne), dt), pltpu.VMEM((ns, ne), dt))


def _body(nin, *args):
    refs = list(args[:nin])
    out_q, out_a, out_c = args[nin:nin + 3]
    hb0, hb1, lc_q, lc_a, lc_c = args[nin + 3:]
    it = iter(refs)

    def nxt():
        return next(it)

    xs = {"question": nxt(), "answer": nxt(), "concept": nxt()}
    lin = {t: (nxt(), nxt(), nxt(), nxt())
           for t in ("question", "answer", "concept")}

    layers = []
    for _ in range(2):
        lw = {}
        for t in ("question", "answer", "concept"):
            lw["q_" + t] = (nxt(), nxt())
        for et in ("has", "rev_has", "mentions", "rev_mentions"):
            lw["k_" + et] = (nxt(), nxt())
            lw["v_" + et] = (nxt(), nxt())
        for t in ("question", "answer", "concept"):
            lw["alin_" + t] = (nxt(), nxt())
        for t in ("question", "answer", "concept"):
            lw["alpha_" + t] = nxt()
        layers.append(lw)

    e_has, e_rev_has, e_mentions, e_rev_mentions = nxt(), nxt(), nxt(), nxt()
    _build_lc(e_rev_has, _NQ, _NA, lc_q, 0)
    _build_lc(e_rev_mentions, _NQ, _NC, lc_q, _NA)
    _build_lc(e_has, _NA, _NQ, lc_a, 0)
    _build_lc(e_mentions, _NC, _NQ, lc_c, 0)
    lcs = {"question": lc_q, "answer": lc_a, "concept": lc_c}

    for t in ("question", "answer", "concept"):
        r0, r1 = _ROWS[t]
        w, b, gamma, beta = lin[t]
        y = _dot(xs[t][...], w[...]) + b[...]
        y = jnp.maximum(y, 0.0)
        n = r1 - r0
        mean = jnp.sum(y, axis=0, keepdims=True) * (1.0 / n)
        yc = y - mean
        var = jnp.sum(yc * yc, axis=0, keepdims=True) * (1.0 / n)
        y = yc * jax.lax.rsqrt(var + 1e-5) * gamma[...] + beta[...]
        hb0[r0:r1] = y.astype(_BF16)

    def w0(dst):
        def wr(v):
            r0, r1 = _ROWS[dst]
            hb1[r0:r1] = v.astype(_BF16)
        return wr

    _layer(hb0, layers[0], lcs,
           {d: w0(d) for d in ("question", "answer", "concept")})

    outs = {"question": out_q, "answer": out_a, "concept": out_c}

    def w1(dst):
        def wr(v):
            outs[dst][...] = v
        return wr

    _layer(hb1, layers[1], lcs,
           {d: w1(d) for d in ("question", "answer", "concept")})


def _fold(w, b, rel):
    wf = jnp.einsum("chd,hde->che", w.reshape(_CH, _H, _HD), rel,
                    preferred_element_type=jnp.float32).reshape(_CH, _CH)
    bf = jnp.einsum("hd,hde->he", b.reshape(_H, _HD), rel,
                    preferred_element_type=jnp.float32).reshape(1, _CH)
    return wf.astype(_BF16), bf


def kernel(lin_w_question, lin_b_question, bn_gamma_question, bn_beta_question, lin_w_answer, lin_b_answer, bn_gamma_answer, bn_beta_answer, lin_w_concept, lin_b_concept, bn_gamma_concept, bn_beta_concept, c0_k_w_question, c0_k_b_question, c0_q_w_question, c0_q_b_question, c0_v_w_question, c0_v_b_question, c0_alin_w_question, c0_alin_b_question, c0_skip_question, c0_k_w_answer, c0_k_b_answer, c0_q_w_answer, c0_q_b_answer, c0_v_w_answer, c0_v_b_answer, c0_alin_w_answer, c0_alin_b_answer, c0_skip_answer, c0_k_w_concept, c0_k_b_concept, c0_q_w_concept, c0_q_b_concept, c0_v_w_concept, c0_v_b_concept, c0_alin_w_concept, c0_alin_b_concept, c0_skip_concept, c0_arel_question_has_answer, c0_mrel_question_has_answer, c0_prel_question_has_answer, c0_arel_answer_rev_has_question, c0_mrel_answer_rev_has_question, c0_prel_answer_rev_has_question, c0_arel_question_mentions_concept, c0_mrel_question_mentions_concept, c0_prel_question_mentions_concept, c0_arel_concept_rev_mentions_question, c0_mrel_concept_rev_mentions_question, c0_prel_concept_rev_mentions_question, c1_k_w_question, c1_k_b_question, c1_q_w_question, c1_q_b_question, c1_v_w_question, c1_v_b_question, c1_alin_w_question, c1_alin_b_question, c1_skip_question, c1_k_w_answer, c1_k_b_answer, c1_q_w_answer, c1_q_b_answer, c1_v_w_answer, c1_v_b_answer, c1_alin_w_answer, c1_alin_b_answer, c1_skip_answer, c1_k_w_concept, c1_k_b_concept, c1_q_w_concept, c1_q_b_concept, c1_v_w_concept, c1_v_b_concept, c1_alin_w_concept, c1_alin_b_concept, c1_skip_concept, c1_arel_question_has_answer, c1_mrel_question_has_answer, c1_prel_question_has_answer, c1_arel_answer_rev_has_question, c1_mrel_answer_rev_has_question, c1_prel_answer_rev_has_question, c1_arel_question_mentions_concept, c1_mrel_question_mentions_concept, c1_prel_question_mentions_concept, c1_arel_concept_rev_mentions_question, c1_mrel_concept_rev_mentions_question, c1_prel_concept_rev_mentions_question, x_question, x_answer, x_concept, edge_question_has_answer, edge_answer_rev_has_question, edge_question_mentions_concept, edge_concept_rev_mentions_question):
    c0 = {
        "k_w": {"question": c0_k_w_question, "answer": c0_k_w_answer, "concept": c0_k_w_concept},
        "k_b": {"question": c0_k_b_question, "answer": c0_k_b_answer, "concept": c0_k_b_concept},
        "q_w": {"question": c0_q_w_question, "answer": c0_q_w_answer, "concept": c0_q_w_concept},
        "q_b": {"question": c0_q_b_question, "answer": c0_q_b_answer, "concept": c0_q_b_concept},
        "v_w": {"question": c0_v_w_question, "answer": c0_v_w_answer, "concept": c0_v_w_concept},
        "v_b": {"question": c0_v_b_question, "answer": c0_v_b_answer, "concept": c0_v_b_concept},
        "alin_w": {"question": c0_alin_w_question, "answer": c0_alin_w_answer, "concept": c0_alin_w_concept},
        "alin_b": {"question": c0_alin_b_question, "answer": c0_alin_b_answer, "concept": c0_alin_b_concept},
        "skip": {"question": c0_skip_question, "answer": c0_skip_answer, "concept": c0_skip_concept},
        "arel": {"has": c0_arel_question_has_answer, "rev_has": c0_arel_answer_rev_has_question,
                 "mentions": c0_arel_question_mentions_concept, "rev_mentions": c0_arel_concept_rev_mentions_question},
        "mrel": {"has": c0_mrel_question_has_answer, "rev_has": c0_mrel_answer_rev_has_question,
                 "mentions": c0_mrel_question_mentions_concept, "rev_mentions": c0_mrel_concept_rev_mentions_question},
        "prel": {"has": c0_prel_question_has_answer, "rev_has": c0_prel_answer_rev_has_question,
                 "mentions": c0_prel_question_mentions_concept, "rev_mentions": c0_prel_concept_rev_mentions_question},
    }
    c1 = {
        "k_w": {"question": c1_k_w_question, "answer": c1_k_w_answer, "concept": c1_k_w_concept},
        "k_b": {"question": c1_k_b_question, "answer": c1_k_b_answer, "concept": c1_k_b_concept},
        "q_w": {"question": c1_q_w_question, "answer": c1_q_w_answer, "concept": c1_q_w_concept},
        "q_b": {"question": c1_q_b_question, "answer": c1_q_b_answer, "concept": c1_q_b_concept},
        "v_w": {"question": c1_v_w_question, "answer": c1_v_w_answer, "concept": c1_v_w_concept},
        "v_b": {"question": c1_v_b_question, "answer": c1_v_b_answer, "concept": c1_v_b_concept},
        "alin_w": {"question": c1_alin_w_question, "answer": c1_alin_w_answer, "concept": c1_alin_w_concept},
        "alin_b": {"question": c1_alin_b_question, "answer": c1_alin_b_answer, "concept": c1_alin_b_concept},
        "skip": {"question": c1_skip_question, "answer": c1_skip_answer, "concept": c1_skip_concept},
        "arel": {"has": c1_arel_question_has_answer, "rev_has": c1_arel_answer_rev_has_question,
                 "mentions": c1_arel_question_mentions_concept, "rev_mentions": c1_arel_concept_rev_mentions_question},
        "mrel": {"has": c1_mrel_question_has_answer, "rev_has": c1_mrel_answer_rev_has_question,
                 "mentions": c1_mrel_question_mentions_concept, "rev_mentions": c1_mrel_concept_rev_mentions_question},
        "prel": {"has": c1_prel_question_has_answer, "rev_has": c1_prel_answer_rev_has_question,
                 "mentions": c1_prel_question_mentions_concept, "rev_mentions": c1_prel_concept_rev_mentions_question},
    }
    et_src = {"has": "question", "rev_has": "answer",
              "mentions": "question", "rev_mentions": "concept"}

    ins = [x_question.astype(_BF16), x_answer.astype(_BF16), x_concept.astype(_BF16)]
    lin_w = {"question": lin_w_question, "answer": lin_w_answer, "concept": lin_w_concept}
    lin_b = {"question": lin_b_question, "answer": lin_b_answer, "concept": lin_b_concept}
    bn_g = {"question": bn_gamma_question, "answer": bn_gamma_answer, "concept": bn_gamma_concept}
    bn_b = {"question": bn_beta_question, "answer": bn_beta_answer, "concept": bn_beta_concept}
    for t in ("question", "answer", "concept"):
        ins += [lin_w[t].astype(_BF16), lin_b[t], bn_g[t], bn_b[t]]

    for cl in (c0, c1):
        for t in ("question", "answer", "concept"):
            ins += [cl["q_w"][t].astype(_BF16), cl["q_b"][t]]
        for et in ("has", "rev_has", "mentions", "rev_mentions"):
            s = et_src[et]
            ka = cl["arel"][et] * (cl["prel"][et] / math.sqrt(_HD))[:, None, None]
            kw, kb = _fold(cl["k_w"][s], cl["k_b"][s], ka)
            vw, vb = _fold(cl["v_w"][s], cl["v_b"][s], cl["mrel"][et])
            ins += [kw, kb, vw, vb]
        for t in ("question", "answer", "concept"):
            ins += [cl["alin_w"][t].astype(_BF16), cl["alin_b"][t]]
        for t in ("question", "answer", "concept"):
            ins.append(jnp.broadcast_to(jax.nn.sigmoid(cl["skip"][t]),
                                        (1, _CH)).astype(jnp.float32))

    ins += [edge_question_has_answer, edge_answer_rev_has_question,
            edge_question_mentions_concept, edge_concept_rev_mentions_question]

    nin = len(ins)
    out = pl.pallas_call(
        functools.partial(_body, nin),
        out_shape=(jax.ShapeDtypeStruct((_NQ, _CH), jnp.float32),
                   jax.ShapeDtypeStruct((_NA, _CH), jnp.float32),
                   jax.ShapeDtypeStruct((_NC, _CH), jnp.float32)),
        in_specs=[pl.BlockSpec(memory_space=pltpu.MemorySpace.VMEM)] * nin,
        out_specs=(pl.BlockSpec(memory_space=pltpu.MemorySpace.VMEM),) * 3,
        scratch_shapes=[pltpu.VMEM((_NTOT, _CH), _BF16),
                        pltpu.VMEM((_NTOT, _CH), _BF16),
                        pltpu.VMEM((_NQ, _NA + _NC), jnp.float32),
                        pltpu.VMEM((_NA, _NQ), jnp.float32),
                        pltpu.VMEM((_NC, _NQ), jnp.float32)],
        compiler_params=pltpu.CompilerParams(
            vmem_limit_bytes=56 * 1024 * 1024),
    )(*ins)
    return {"question": out[0], "answer": out[1], "concept": out[2]}
